# baseline (speedup 1.0000x reference)
import functools

import jax
import jax.numpy as jnp
from jax import lax
from jax.experimental import pallas as pl
from jax.experimental.pallas import tpu as pltpu
from jax.experimental.pallas import tpu_sc as plsc

_LANES = 16
_NC = 2
_NS = 16
_BINS = 52
_UNROLL = 16
_NCHUNK = 8


@functools.partial(jax.jit, static_argnums=(0,))
def _histogram_embed(n, cards, rank_embed, suit_embed):
    nw = _NC * _NS
    chunk = n // nw
    csub = chunk // _NCHUNK
    vec_sub = csub // _LANES
    assert n == chunk * nw and csub % (_UNROLL * _LANES) == 0

    mesh = plsc.VectorSubcoreMesh(
        core_axis_name="c", subcore_axis_name="s",
        num_cores=_NC, num_subcores=_NS)

    @functools.partial(
        pl.kernel,
        out_type=jax.ShapeDtypeStruct((nw, _LANES), jnp.float32),
        mesh=mesh,
        compiler_params=pltpu.CompilerParams(needs_layout_passes=False),
        scratch_types=[
            pltpu.VMEM((csub,), jnp.int32),
            pltpu.VMEM((csub,), jnp.int32),
            pltpu.VMEM((_BINS * _LANES,), jnp.float32),
            pltpu.VMEM((13, 8), jnp.float32),
            pltpu.VMEM((4, 4), jnp.float32),
            pltpu.VMEM((_LANES,), jnp.float32),
            pltpu.SemaphoreType.DMA,
            pltpu.SemaphoreType.DMA,
        ],
    )
    def card_embed(cards_hbm, re_hbm, se_hbm, out_hbm,
                   buf0, buf1, hist_v, re_v, se_v, acc_v, sem0, sem1):
        cid = lax.axis_index("c")
        sid = lax.axis_index("s")
        wid = sid * _NC + cid
        base = wid * chunk

        pltpu.async_copy(cards_hbm.at[pl.ds(base, csub)], buf0, sem0)
        pltpu.async_copy(cards_hbm.at[pl.ds(base + csub, csub)], buf1, sem1)

        pltpu.sync_copy(re_hbm, re_v)
        pltpu.sync_copy(se_hbm, se_v)
        zeros = jnp.zeros((_LANES,), jnp.float32)

        def zbody(b, carry):
            hist_v[pl.ds(b * _LANES, _LANES)] = zeros
            return carry

        lax.fori_loop(0, _BINS, zbody, 0)

        lanes = lax.broadcasted_iota(jnp.int32, (_LANES,), 0)
        ones = jnp.ones((_LANES,), jnp.float32)
        npair = _NCHUNK // 2

        def pair_body(j, carry):
            for b, (bv, sem) in enumerate(((buf0, sem0), (buf1, sem1))):
                pltpu.make_async_copy(
                    cards_hbm.at[pl.ds(base, csub)], bv, sem).wait()

                @plsc.parallel_loop(0, vec_sub, step=1, unroll=_UNROLL)
                def _hist(i, bv=bv):
                    c = bv[pl.ds(i * _LANES, _LANES)]
                    plsc.addupdate_scatter(hist_v, [c * _LANES + lanes], ones)

                @pl.when(j + 1 < npair)
                def _():
                    nxt = base + (2 * (j + 1) + b) * csub
                    pltpu.async_copy(cards_hbm.at[pl.ds(nxt, csub)], bv, sem)
            return carry

        lax.fori_loop(0, npair, pair_body, 0)

        mask_r = lanes < 8
        mask_s = jnp.logical_and(lanes >= 8, lanes < 12)
        lanes7 = jnp.bitwise_and(lanes, 7)
        lanes3 = jnp.bitwise_and(lanes, 3)

        @plsc.parallel_loop(0, _BINS, step=1, unroll=4, carry=zeros)
        def acc(b, acc_c):
            cnt = jnp.sum(hist_v[pl.ds(b * _LANES, _LANES)])
            r = b % 13
            s = b // 13
            g_r = plsc.load_gather(re_v, [jnp.full((_LANES,), r, jnp.int32), lanes7])
            g_s = plsc.load_gather(se_v, [jnp.full((_LANES,), s, jnp.int32), lanes3])
            w = jnp.where(mask_r, g_r, zeros) + jnp.where(mask_s, g_s, zeros)
            return acc_c + cnt * w
        acc_v[...] = acc * jnp.float32(1.0 / n)
        pltpu.sync_copy(acc_v, out_hbm.at[wid])

    return card_embed(cards, rank_embed, suit_embed)


def kernel(cards, rank_embed, suit_embed):
    n = cards.shape[0]
    cards = cards.astype(jnp.int32)
    parts = _histogram_embed(n, cards, rank_embed, suit_embed)
    return jnp.sum(parts, axis=0)[:12]


# speedup vs baseline: 1.0137x; 1.0137x over previous
import functools

import jax
import jax.numpy as jnp
from jax import lax
from jax.experimental import pallas as pl
from jax.experimental.pallas import tpu as pltpu
from jax.experimental.pallas import tpu_sc as plsc

_LANES = 16
_NC = 2
_NS = 16
_BINS = 52
_UNROLL = 8
_NCHUNK = 4


@functools.partial(jax.jit, static_argnums=(0,))
def _histogram_embed(n, cards, rank_embed, suit_embed):
    nw = _NC * _NS
    chunk = n // nw
    csub = chunk // _NCHUNK
    vec_sub = csub // _LANES
    assert n == chunk * nw and csub % (_UNROLL * _LANES) == 0

    mesh = plsc.VectorSubcoreMesh(
        core_axis_name="c", subcore_axis_name="s",
        num_cores=_NC, num_subcores=_NS)

    @functools.partial(
        pl.kernel,
        out_type=jax.ShapeDtypeStruct((nw, _LANES), jnp.float32),
        mesh=mesh,
        compiler_params=pltpu.CompilerParams(needs_layout_passes=False),
        scratch_types=[
            pltpu.VMEM((csub,), jnp.int32),
            pltpu.VMEM((csub,), jnp.int32),
            pltpu.VMEM((_BINS * _LANES,), jnp.float32),
            pltpu.VMEM((13, 8), jnp.float32),
            pltpu.VMEM((4, 4), jnp.float32),
            pltpu.VMEM((_LANES,), jnp.float32),
            pltpu.SemaphoreType.DMA,
            pltpu.SemaphoreType.DMA,
        ],
    )
    def card_embed(cards_hbm, re_hbm, se_hbm, out_hbm,
                   buf0, buf1, hist_v, re_v, se_v, acc_v, sem0, sem1):
        cid = lax.axis_index("c")
        sid = lax.axis_index("s")
        wid = sid * _NC + cid
        base = wid * chunk

        pltpu.async_copy(cards_hbm.at[pl.ds(base, csub)], buf0, sem0)
        pltpu.async_copy(cards_hbm.at[pl.ds(base + csub, csub)], buf1, sem1)

        pltpu.sync_copy(re_hbm, re_v)
        pltpu.sync_copy(se_hbm, se_v)
        zeros = jnp.zeros((_LANES,), jnp.float32)

        def zbody(b, carry):
            hist_v[pl.ds(b * _LANES, _LANES)] = zeros
            return carry

        lax.fori_loop(0, _BINS, zbody, 0)

        lanes = lax.broadcasted_iota(jnp.int32, (_LANES,), 0)
        ones = jnp.ones((_LANES,), jnp.float32)
        npair = _NCHUNK // 2

        def pair_body(j, carry):
            for b, (bv, sem) in enumerate(((buf0, sem0), (buf1, sem1))):
                pltpu.make_async_copy(
                    cards_hbm.at[pl.ds(base, csub)], bv, sem).wait()

                @plsc.parallel_loop(0, vec_sub, step=1, unroll=_UNROLL)
                def _hist(i, bv=bv):
                    c = bv[pl.ds(i * _LANES, _LANES)]
                    plsc.addupdate_scatter(hist_v, [c * _LANES + lanes], ones)

                @pl.when(j + 1 < npair)
                def _():
                    nxt = base + (2 * (j + 1) + b) * csub
                    pltpu.async_copy(cards_hbm.at[pl.ds(nxt, csub)], bv, sem)
            return carry

        lax.fori_loop(0, npair, pair_body, 0)

        mask_r = lanes < 8
        mask_s = jnp.logical_and(lanes >= 8, lanes < 12)
        lanes7 = jnp.bitwise_and(lanes, 7)
        lanes3 = jnp.bitwise_and(lanes, 3)

        @plsc.parallel_loop(0, _BINS, step=1, unroll=4, carry=zeros)
        def acc(b, acc_c):
            cnt = jnp.sum(hist_v[pl.ds(b * _LANES, _LANES)])
            r = b % 13
            s = b // 13
            g_r = plsc.load_gather(re_v, [jnp.full((_LANES,), r, jnp.int32), lanes7])
            g_s = plsc.load_gather(se_v, [jnp.full((_LANES,), s, jnp.int32), lanes3])
            w = jnp.where(mask_r, g_r, zeros) + jnp.where(mask_s, g_s, zeros)
            return acc_c + cnt * w
        acc_v[...] = acc * jnp.float32(1.0 / n)
        pltpu.sync_copy(acc_v, out_hbm.at[wid])

    return card_embed(cards, rank_embed, suit_embed)


def kernel(cards, rank_embed, suit_embed):
    n = cards.shape[0]
    cards = cards.astype(jnp.int32)
    parts = _histogram_embed(n, cards, rank_embed, suit_embed)
    return jnp.sum(parts, axis=0)[:12]


# speedup vs baseline: 1.0339x; 1.0200x over previous
import functools

import jax
import jax.numpy as jnp
from jax import lax
from jax.experimental import pallas as pl
from jax.experimental.pallas import tpu as pltpu
from jax.experimental.pallas import tpu_sc as plsc

_LANES = 16
_NC = 2
_NS = 16
_BINS = 52
_UNROLL = 8
_NCHUNK = 4


@functools.partial(jax.jit, static_argnums=(0,))
def _histogram_embed(n, cards, tables):
    nw = _NC * _NS
    chunk = n // nw
    csub = chunk // _NCHUNK
    vec_sub = csub // _LANES
    assert n == chunk * nw and csub % (_UNROLL * _LANES) == 0

    mesh = plsc.VectorSubcoreMesh(
        core_axis_name="c", subcore_axis_name="s",
        num_cores=_NC, num_subcores=_NS)

    @functools.partial(
        pl.kernel,
        out_type=jax.ShapeDtypeStruct((nw, _LANES), jnp.float32),
        mesh=mesh,
        compiler_params=pltpu.CompilerParams(needs_layout_passes=False),
        scratch_types=[
            pltpu.VMEM((csub,), jnp.int32),
            pltpu.VMEM((csub,), jnp.int32),
            pltpu.VMEM((_BINS * _LANES,), jnp.float32),
            pltpu.VMEM((128,), jnp.float32),
            pltpu.VMEM((_LANES,), jnp.float32),
            pltpu.SemaphoreType.DMA,
            pltpu.SemaphoreType.DMA,
        ],
    )
    def card_embed(cards_hbm, tf_hbm, out_hbm,
                   buf0, buf1, hist_v, tf_v, acc_v, sem0, sem1):
        cid = lax.axis_index("c")
        sid = lax.axis_index("s")
        wid = sid * _NC + cid
        base = wid * chunk

        pltpu.async_copy(cards_hbm.at[pl.ds(base, csub)], buf0, sem0)
        pltpu.async_copy(cards_hbm.at[pl.ds(base + csub, csub)], buf1, sem1)

        pltpu.sync_copy(tf_hbm, tf_v)
        zeros = jnp.zeros((_LANES,), jnp.float32)

        def zbody(b, carry):
            hist_v[pl.ds(b * _LANES, _LANES)] = zeros
            return carry

        lax.fori_loop(0, _BINS, zbody, 0)

        lanes = lax.broadcasted_iota(jnp.int32, (_LANES,), 0)
        ones = jnp.ones((_LANES,), jnp.float32)
        npair = _NCHUNK // 2

        def pair_body(j, carry):
            for b, (bv, sem) in enumerate(((buf0, sem0), (buf1, sem1))):
                pltpu.make_async_copy(
                    cards_hbm.at[pl.ds(base, csub)], bv, sem).wait()

                @plsc.parallel_loop(0, vec_sub, step=1, unroll=_UNROLL)
                def _hist(i, bv=bv):
                    c = bv[pl.ds(i * _LANES, _LANES)]
                    plsc.addupdate_scatter(hist_v, [c * _LANES + lanes], ones)

                @pl.when(j + 1 < npair)
                def _():
                    nxt = base + (2 * (j + 1) + b) * csub
                    pltpu.async_copy(cards_hbm.at[pl.ds(nxt, csub)], bv, sem)
            return carry

        lax.fori_loop(0, npair, pair_body, 0)

        mask_r = lanes < 8
        mask_s = jnp.logical_and(lanes >= 8, lanes < 12)
        lanes7 = jnp.bitwise_and(lanes, 7)
        lanes3 = jnp.bitwise_and(lanes, 3)

        @plsc.parallel_loop(0, _BINS, step=1, unroll=4, carry=zeros)
        def acc(b, acc_c):
            cnt = jnp.sum(hist_v[pl.ds(b * _LANES, _LANES)])
            r = b % 13
            s = b // 13
            g_r = plsc.load_gather(tf_v, [r * 8 + lanes7])
            g_s = plsc.load_gather(tf_v, [104 + s * 4 + lanes3])
            w = jnp.where(mask_r, g_r, zeros) + jnp.where(mask_s, g_s, zeros)
            return acc_c + cnt * w
        acc_v[...] = acc * jnp.float32(1.0 / n)
        pltpu.sync_copy(acc_v, out_hbm.at[wid])

    return card_embed(cards, tables)


def kernel(cards, rank_embed, suit_embed):
    n = cards.shape[0]
    cards = cards.astype(jnp.int32)
    tables = jnp.concatenate([rank_embed.reshape(-1), suit_embed.reshape(-1),
                              jnp.zeros((8,), jnp.float32)])
    parts = _histogram_embed(n, cards, tables)
    return jnp.sum(parts, axis=0)[:12]
